# baseline (speedup 1.0000x reference)
import functools

import jax
import jax.numpy as jnp
from jax import lax
from jax.experimental import pallas as pl
from jax.experimental.pallas import tpu as pltpu
from jax.experimental.pallas import tpu_sc as plsc

VOCAB = 1000000
SEQ = 200
D = 64
LANES = 16
NC, NS = 2, 16
NW = NC * NS


def _build(total_rows: int):
    per_w = total_rows // NW
    chunk = 1600
    n_chunks = per_w // chunk
    reps = chunk // SEQ

    mesh = plsc.VectorSubcoreMesh(core_axis_name="c", subcore_axis_name="s")

    @functools.partial(
        pl.kernel,
        out_type=jax.ShapeDtypeStruct((total_rows, D), jnp.float32),
        mesh=mesh,
        scratch_types=[
            pltpu.VMEM((chunk,), jnp.int32),
            pltpu.VMEM((chunk, D), jnp.float32),
            pltpu.VMEM((SEQ, D), jnp.float32),
            pltpu.SemaphoreType.DMA,
        ],
        compiler_params=pltpu.CompilerParams(use_tc_tiling_on_sc=False),
    )
    def k(idx_hbm, tok_hbm, pos_hbm, out_hbm, idx_v, rows_v, pos_v, sem):
        wid = lax.axis_index("s") * NC + lax.axis_index("c")
        base = wid * per_w
        pltpu.sync_copy(pos_hbm, pos_v)

        def chunk_body(g, carry):
            start = base + g * chunk
            pltpu.sync_copy(idx_hbm.at[pl.ds(start, chunk)], idx_v)
            pltpu.async_copy(tok_hbm.at[idx_v], rows_v, sem).wait()

            def add_body(s, c2):
                for j in range(D // LANES):
                    pvec = pos_v[s, pl.ds(j * LANES, LANES)]
                    for q in range(reps):
                        plsc.addupdate(
                            rows_v.at[q * SEQ + s, pl.ds(j * LANES, LANES)], pvec
                        )
                return c2

            lax.fori_loop(0, SEQ, add_body, 0, unroll=False)
            pltpu.sync_copy(rows_v, out_hbm.at[pl.ds(start, chunk)])
            return carry

        lax.fori_loop(0, n_chunks, chunk_body, 0, unroll=False)

    return k


def kernel(inputs, token_table, position_table):
    b, s = inputs.shape
    total = b * s
    idx_flat = jnp.reshape(inputs, (total,)).astype(jnp.int32)
    out = _build(total)(idx_flat, token_table, position_table)
    return jnp.reshape(out, (b, s, D))
